# baseline (speedup 1.0000x reference)
import jax
import jax.numpy as jnp
from jax.experimental import pallas as pl
from jax.experimental.pallas import tpu as pltpu

N = 10000
H = 128
NCLASSES = 40
BM = 200
NBUF = 5
NI = N // BM


def _in_mlp_kernel(f_ref, w1_ref, b1_ref, o_ref):
    o_ref[...] = jax.nn.relu(
        jnp.dot(f_ref[...], w1_ref[...]) + b1_ref[...]
    )


def _gru_ginmlp(y, h, wih, whh, bih, bhh, wg1, bg1, wg2, bg2):
    gi = jnp.dot(y, wih) + bih
    gh = jnp.dot(h, whh) + bhh
    r = jax.nn.sigmoid(gi[:, :H] + gh[:, :H])
    z = jax.nn.sigmoid(gi[:, H:2 * H] + gh[:, H:2 * H])
    n = jnp.tanh(gi[:, 2 * H:] + r * gh[:, 2 * H:])
    hn = (1.0 - z) * n + z * h
    g = jnp.dot(jax.nn.relu(jnp.dot(hn, wg1) + bg1), wg2) + bg2
    return jax.nn.relu(g)


def _block_copy(adj_hbm, buf, sem, blk):
    return pltpu.make_async_copy(
        adj_hbm.at[pl.ds(blk * BM, BM), :], buf.at[blk % NBUF],
        sem.at[blk % NBUF])


def _stream_spmm(i, adj_hbm, x_ref, buf, sem):
    @pl.when(i == 0)
    def _():
        for j in range(NBUF - 1):
            _block_copy(adj_hbm, buf, sem, j).start()

    nxt = i + NBUF - 1

    @pl.when(nxt < NI)
    def _():
        _block_copy(adj_hbm, buf, sem, nxt).start()

    _block_copy(adj_hbm, buf, sem, i).wait()
    return jnp.dot(buf[i % NBUF], x_ref[...],
                   preferred_element_type=jnp.float32)


def _layer_kernel(adj_hbm, x_ref, wih_ref, whh_ref, bih_ref,
                  bhh_ref, wg1_ref, bg1_ref, wg2_ref, bg2_ref,
                  o_ref, buf, sem):
    i = pl.program_id(0)
    y = _stream_spmm(i, adj_hbm, x_ref, buf, sem)
    h = x_ref[pl.ds(i * BM, BM), :]
    o_ref[...] = _gru_ginmlp(y, h, wih_ref[...], whh_ref[...],
                             bih_ref[...], bhh_ref[...], wg1_ref[...],
                             bg1_ref[...], wg2_ref[...], bg2_ref[...])


def _final_kernel(adj_hbm, x_ref, wih_ref, whh_ref, bih_ref,
                  bhh_ref, wg1_ref, bg1_ref, wg2_ref, bg2_ref,
                  wc_ref, bc_ref, wd_ref, bd_ref, pred_ref, buf, sem):
    i = pl.program_id(0)
    y = _stream_spmm(i, adj_hbm, x_ref, buf, sem)
    h = x_ref[pl.ds(i * BM, BM), :]
    out = _gru_ginmlp(y, h, wih_ref[...], whh_ref[...],
                      bih_ref[...], bhh_ref[...], wg1_ref[...],
                      bg1_ref[...], wg2_ref[...], bg2_ref[...])
    t = jnp.dot(jax.nn.relu(jnp.dot(out, wc_ref[...]) + bc_ref[...]),
                wd_ref[...]) + bd_ref[...]
    m = jnp.max(t, axis=1, keepdims=True)
    e = jnp.exp(t - m)
    pred_ref[...] = e / jnp.sum(e, axis=1, keepdims=True)


def _full(shape):
    return pl.BlockSpec(shape, lambda i: (0,) * len(shape))


def _layer_call(adj, x, wih_t, whh_t, bih, bhh, wg1, bg1,
                wg2, bg2, head=None):
    weight_specs = [
        _full((H, 3 * H)), _full((H, 3 * H)), _full((1, 3 * H)),
        _full((1, 3 * H)), _full((H, H)), _full((1, H)),
        _full((H, H)), _full((1, H)),
    ]
    in_specs = [
        pl.BlockSpec(memory_space=pl.ANY),
        _full((N, H)),
    ] + weight_specs
    args = [adj, x, wih_t, whh_t, bih, bhh, wg1, bg1, wg2, bg2]
    if head is None:
        body = _layer_kernel
        out_shape = jax.ShapeDtypeStruct((N, H), jnp.float32)
        out_specs = pl.BlockSpec((BM, H), lambda i: (i, 0))
    else:
        body = _final_kernel
        wc, bc, wd, bd = head
        in_specs += [_full((H, H)), _full((1, H)),
                     _full((H, NCLASSES)), _full((1, NCLASSES))]
        args += [wc, bc, wd, bd]
        out_shape = jax.ShapeDtypeStruct((N, NCLASSES), jnp.float32)
        out_specs = pl.BlockSpec((BM, NCLASSES), lambda i: (i, 0))
    return pl.pallas_call(
        body,
        grid=(NI,),
        in_specs=in_specs,
        out_specs=out_specs,
        out_shape=out_shape,
        scratch_shapes=[
            pltpu.VMEM((NBUF, BM, N), jnp.float32),
            pltpu.SemaphoreType.DMA((NBUF,)),
        ],
        compiler_params=pltpu.CompilerParams(
            dimension_semantics=("arbitrary",)),
    )(*args)


def kernel(features, adj, W1, b1, Wih, Whh, bih, bhh, Wg1, bg1, Wg2, bg2,
           Wc, bc, Wd, bd):
    x = pl.pallas_call(
        _in_mlp_kernel,
        grid=(10,),
        in_specs=[pl.BlockSpec((N // 10, H), lambda i: (i, 0)),
                  pl.BlockSpec((H, H), lambda i: (0, 0)),
                  pl.BlockSpec((1, H), lambda i: (0, 0))],
        out_specs=pl.BlockSpec((N // 10, H), lambda i: (i, 0)),
        out_shape=jax.ShapeDtypeStruct((N, H), jnp.float32),
        compiler_params=pltpu.CompilerParams(
            dimension_semantics=("arbitrary",)),
    )(features, W1, b1.reshape(1, H))

    for i in range(2):
        layer = dict(
            wih_t=Wih[i].T, whh_t=Whh[i].T,
            bih=bih[i].reshape(1, 3 * H), bhh=bhh[i].reshape(1, 3 * H),
            wg1=Wg1[i], bg1=bg1[i].reshape(1, H),
            wg2=Wg2[i], bg2=bg2[i].reshape(1, H),
        )
        if i == 0:
            x = _layer_call(adj, x, **layer)
        else:
            preds = _layer_call(
                adj, x, **layer,
                head=(Wc, bc.reshape(1, H), Wd, bd.reshape(1, NCLASSES)))
    return preds


# speedup vs baseline: 1.0252x; 1.0252x over previous
import jax
import jax.numpy as jnp
from jax.experimental import pallas as pl
from jax.experimental.pallas import tpu as pltpu

N = 10000
H = 128
NCLASSES = 40
BM = 400
NBUF = 2
NI = N // BM


def _in_mlp_kernel(f_ref, w1_ref, b1_ref, o_ref):
    o_ref[...] = jax.nn.relu(
        jnp.dot(f_ref[...], w1_ref[...]) + b1_ref[...]
    )


def _gru_ginmlp(y, h, wih, whh, bih, bhh, wg1, bg1, wg2, bg2):
    gi = jnp.dot(y, wih) + bih
    gh = jnp.dot(h, whh) + bhh
    r = jax.nn.sigmoid(gi[:, :H] + gh[:, :H])
    z = jax.nn.sigmoid(gi[:, H:2 * H] + gh[:, H:2 * H])
    n = jnp.tanh(gi[:, 2 * H:] + r * gh[:, 2 * H:])
    hn = (1.0 - z) * n + z * h
    g = jnp.dot(jax.nn.relu(jnp.dot(hn, wg1) + bg1), wg2) + bg2
    return jax.nn.relu(g)


HB = BM // 2


def _half_copy(adj_hbm, buf, sem, blk, half):
    rows = blk * BM + half * HB
    return pltpu.make_async_copy(
        adj_hbm.at[pl.ds(rows, HB), :],
        buf.at[blk % NBUF, pl.ds(half * HB, HB), :],
        sem.at[blk % NBUF, half])


def _start_block(adj_hbm, buf, sem, blk):
    _half_copy(adj_hbm, buf, sem, blk, 0).start()
    _half_copy(adj_hbm, buf, sem, blk, 1).start()


def _stream_spmm(i, adj_hbm, x_ref, buf, sem):
    @pl.when(i == 0)
    def _():
        for j in range(NBUF - 1):
            _start_block(adj_hbm, buf, sem, j)

    nxt = i + NBUF - 1

    @pl.when(nxt < NI)
    def _():
        _start_block(adj_hbm, buf, sem, nxt)

    _half_copy(adj_hbm, buf, sem, i, 0).wait()
    _half_copy(adj_hbm, buf, sem, i, 1).wait()
    return jnp.dot(buf[i % NBUF], x_ref[...],
                   preferred_element_type=jnp.float32)


def _layer_kernel(adj_hbm, x_ref, wih_ref, whh_ref, bih_ref,
                  bhh_ref, wg1_ref, bg1_ref, wg2_ref, bg2_ref,
                  o_ref, buf, sem):
    i = pl.program_id(0)
    y = _stream_spmm(i, adj_hbm, x_ref, buf, sem)
    h = x_ref[pl.ds(i * BM, BM), :]
    o_ref[...] = _gru_ginmlp(y, h, wih_ref[...], whh_ref[...],
                             bih_ref[...], bhh_ref[...], wg1_ref[...],
                             bg1_ref[...], wg2_ref[...], bg2_ref[...])


def _final_kernel(adj_hbm, x_ref, wih_ref, whh_ref, bih_ref,
                  bhh_ref, wg1_ref, bg1_ref, wg2_ref, bg2_ref,
                  wc_ref, bc_ref, wd_ref, bd_ref, pred_ref, buf, sem):
    i = pl.program_id(0)
    y = _stream_spmm(i, adj_hbm, x_ref, buf, sem)
    h = x_ref[pl.ds(i * BM, BM), :]
    out = _gru_ginmlp(y, h, wih_ref[...], whh_ref[...],
                      bih_ref[...], bhh_ref[...], wg1_ref[...],
                      bg1_ref[...], wg2_ref[...], bg2_ref[...])
    t = jnp.dot(jax.nn.relu(jnp.dot(out, wc_ref[...]) + bc_ref[...]),
                wd_ref[...]) + bd_ref[...]
    m = jnp.max(t, axis=1, keepdims=True)
    e = jnp.exp(t - m)
    pred_ref[...] = e / jnp.sum(e, axis=1, keepdims=True)


def _full(shape):
    return pl.BlockSpec(shape, lambda i: (0,) * len(shape))


def _layer_call(adj, x, wih_t, whh_t, bih, bhh, wg1, bg1,
                wg2, bg2, head=None):
    weight_specs = [
        _full((H, 3 * H)), _full((H, 3 * H)), _full((1, 3 * H)),
        _full((1, 3 * H)), _full((H, H)), _full((1, H)),
        _full((H, H)), _full((1, H)),
    ]
    in_specs = [
        pl.BlockSpec(memory_space=pl.ANY),
        _full((N, H)),
    ] + weight_specs
    args = [adj, x, wih_t, whh_t, bih, bhh, wg1, bg1, wg2, bg2]
    if head is None:
        body = _layer_kernel
        out_shape = jax.ShapeDtypeStruct((N, H), jnp.float32)
        out_specs = pl.BlockSpec((BM, H), lambda i: (i, 0))
    else:
        body = _final_kernel
        wc, bc, wd, bd = head
        in_specs += [_full((H, H)), _full((1, H)),
                     _full((H, NCLASSES)), _full((1, NCLASSES))]
        args += [wc, bc, wd, bd]
        out_shape = jax.ShapeDtypeStruct((N, NCLASSES), jnp.float32)
        out_specs = pl.BlockSpec((BM, NCLASSES), lambda i: (i, 0))
    return pl.pallas_call(
        body,
        grid=(NI,),
        in_specs=in_specs,
        out_specs=out_specs,
        out_shape=out_shape,
        scratch_shapes=[
            pltpu.VMEM((NBUF, BM, N), jnp.float32),
            pltpu.SemaphoreType.DMA((NBUF, 2)),
        ],
        compiler_params=pltpu.CompilerParams(
            dimension_semantics=("arbitrary",)),
    )(*args)


def kernel(features, adj, W1, b1, Wih, Whh, bih, bhh, Wg1, bg1, Wg2, bg2,
           Wc, bc, Wd, bd):
    x = pl.pallas_call(
        _in_mlp_kernel,
        grid=(10,),
        in_specs=[pl.BlockSpec((N // 10, H), lambda i: (i, 0)),
                  pl.BlockSpec((H, H), lambda i: (0, 0)),
                  pl.BlockSpec((1, H), lambda i: (0, 0))],
        out_specs=pl.BlockSpec((N // 10, H), lambda i: (i, 0)),
        out_shape=jax.ShapeDtypeStruct((N, H), jnp.float32),
        compiler_params=pltpu.CompilerParams(
            dimension_semantics=("arbitrary",)),
    )(features, W1, b1.reshape(1, H))

    for i in range(2):
        layer = dict(
            wih_t=Wih[i].T, whh_t=Whh[i].T,
            bih=bih[i].reshape(1, 3 * H), bhh=bhh[i].reshape(1, 3 * H),
            wg1=Wg1[i], bg1=bg1[i].reshape(1, H),
            wg2=Wg2[i], bg2=bg2[i].reshape(1, H),
        )
        if i == 0:
            x = _layer_call(adj, x, **layer)
        else:
            preds = _layer_call(
                adj, x, **layer,
                head=(Wc, bc.reshape(1, H), Wd, bd.reshape(1, NCLASSES)))
    return preds
